# Initial kernel scaffold; baseline (speedup 1.0000x reference)
import jax
import jax.numpy as jnp
from jax.experimental import pallas as pl


def kernel(x, edge_hidden, edge_index, W):
    raise NotImplementedError("write your pallas kernel here")


# speedup vs baseline: 14.9657x; 14.9657x over previous
import jax
import jax.numpy as jnp
from jax.experimental import pallas as pl
from jax.experimental.pallas import tpu as pltpu


def _body(x_ref, e_ref, w_ref, out_ref, accx_ref, acce_ref, *, grid, d_in):
    i = pl.program_id(0)

    @pl.when(i == 0)
    def _init():
        accx_ref[...] = jnp.zeros_like(accx_ref)
        acce_ref[...] = jnp.zeros_like(acce_ref)

    accx_ref[...] += jnp.sum(x_ref[...], axis=0, keepdims=True)
    acce_ref[...] += jnp.sum(e_ref[...], axis=0, keepdims=True)

    @pl.when(i == grid - 1)
    def _finish():
        sx = accx_ref[...]
        se = acce_ref[...]
        r = jnp.dot(sx, w_ref[:d_in, :], preferred_element_type=jnp.float32)
        r += jnp.dot(se, w_ref[d_in:, :], preferred_element_type=jnp.float32)
        out_ref[...] = r


def kernel(x, edge_hidden, edge_index, W):
    del edge_index
    n_nodes, d_in = x.shape
    n_edges, d_hid = edge_hidden.shape
    grid = 25
    bx = n_nodes // grid
    be = n_edges // grid

    import functools
    body = functools.partial(_body, grid=grid, d_in=d_in)
    out = pl.pallas_call(
        body,
        grid=(grid,),
        in_specs=[
            pl.BlockSpec((bx, d_in), lambda i: (i, 0)),
            pl.BlockSpec((be, d_hid), lambda i: (i, 0)),
            pl.BlockSpec((d_in + d_hid, d_hid), lambda i: (0, 0)),
        ],
        out_specs=pl.BlockSpec((1, d_hid), lambda i: (0, 0)),
        out_shape=jax.ShapeDtypeStruct((1, d_hid), jnp.float32),
        scratch_shapes=[
            pltpu.VMEM((1, d_in), jnp.float32),
            pltpu.VMEM((1, d_hid), jnp.float32),
        ],
    )(x, edge_hidden, W)
    return out[0]
